# baseline (speedup 1.0000x reference)
import functools

import jax
import jax.numpy as jnp
from jax import lax
from jax.experimental import pallas as pl
from jax.experimental.pallas import tpu as pltpu
from jax.experimental.pallas import tpu_sc as plsc

_N0 = 8192
_N1 = 4096

_NC = 2
_NS = 16
_NW = _NC * _NS
_L = 16

_TC_ROWS = 4608
_SC_ROWS = _N0 - _TC_ROWS
_TC_BR = 512
_ROWS_W = _SC_ROWS // _NW
_CROWS = 2
_NBUF = 8
_NCHUNK = _ROWS_W // _CROWS
_UNROLL = 8
_NCHAIN = 4
_NITER = _N1 // (_L * _UNROLL)

_BIG_I32 = 2**31 - 1

_mesh = plsc.VectorSubcoreMesh(core_axis_name="c", subcore_axis_name="s")


@functools.partial(
    pl.kernel,
    mesh=_mesh,
    out_type=[
        jax.ShapeDtypeStruct((_NW, _L), jnp.float32),
        jax.ShapeDtypeStruct((_NW, _L), jnp.int32),
    ],
    scratch_types=[
        pltpu.VMEM((_NBUF, _CROWS, _N1), jnp.float32),
        pltpu.VMEM((_L,), jnp.float32),
        pltpu.VMEM((_L,), jnp.int32),
    ] + [pltpu.SemaphoreType.DMA] * _NBUF,
    compiler_params=pltpu.CompilerParams(use_tc_tiling_on_sc=True),
)
def _sc_partial_argmax(x_hbm, outv_hbm, outi_hbm, buf, stage_v, stage_i,
                       *sems):
    wid = lax.axis_index("s") * _NC + lax.axis_index("c")
    row_base = _TC_ROWS + wid * _ROWS_W
    iota = lax.iota(jnp.int32, _L)

    def allmax(x):
        for s in (8, 4, 2, 1):
            x = jnp.maximum(x, x.at[iota ^ s].get(mode="promise_in_bounds"))
        return x

    def allmin(x):
        for s in (8, 4, 2, 1):
            x = jnp.minimum(x, x.at[iota ^ s].get(mode="promise_in_bounds"))
        return x

    def start(c, slot):
        return pltpu.async_copy(
            x_hbm.at[pl.ds(row_base + c * _CROWS, _CROWS)], buf.at[slot],
            sems[slot])

    def wait_for(c, slot):
        pltpu.make_async_copy(
            x_hbm.at[pl.ds(row_base + c * _CROWS, _CROWS)], buf.at[slot],
            sems[slot]).wait()

    ninf = jnp.full((_L,), float("-inf"), jnp.float32)

    for c0 in range(_NBUF):
        start(c0, c0)

    def chunk_max(slot):
        acc = (ninf,) * _NCHAIN
        for r in range(_CROWS):
            rbuf = buf.at[slot, r]

            def mbody(i, accs, rbuf=rbuf):
                accs = list(accs)
                off = i * (_L * _UNROLL)
                for j in range(_UNROLL):
                    v = rbuf[pl.ds(off + j * _L, _L)]
                    k = j % _NCHAIN
                    accs[k] = jnp.maximum(accs[k], v)
                return tuple(accs)

            acc = lax.fori_loop(0, _NITER, mbody, acc)
        acc = list(acc)
        n = _NCHAIN
        while n > 1:
            n //= 2
            for k in range(n):
                acc[k] = jnp.maximum(acc[k], acc[k + n])
        return acc[0]

    zeros = jnp.zeros((_L,), jnp.int32)

    def cbody(k, carry):
        bestv, bestc = carry
        for slot in range(_NBUF):
            c = _NBUF * k + slot
            wait_for(c, slot)
            cm = chunk_max(slot)
            nxt = c + _NBUF

            @pl.when(nxt < _NCHUNK)
            def _():
                start(nxt, slot)

            better = cm > bestv
            bestc = jnp.where(better, c, bestc)
            bestv = jnp.maximum(bestv, cm)
        return (bestv, bestc)

    bestv, bestc = lax.fori_loop(0, _NCHUNK // _NBUF, cbody, (ninf, zeros))

    m_best = allmax(bestv)[0]
    mvec = jnp.broadcast_to(m_best, (_L,))
    c_best = allmin(jnp.where(bestv == mvec, bestc, _BIG_I32))[0]

    win_row = row_base + c_best * _CROWS
    pltpu.sync_copy(x_hbm.at[pl.ds(win_row, _CROWS)], buf.at[0])
    mvec = jnp.broadcast_to(m_best, (_L,))
    big = jnp.full((_L,), _BIG_I32, jnp.int32)

    pmin = (big, big)
    for r in range(_CROWS):
        rbuf = buf.at[0, r]
        flat_row = (win_row + r) * _N1

        def rbody(i, pmins, rbuf=rbuf, flat_row=flat_row):
            p0, p1 = pmins
            off = i * (_L * _UNROLL)
            for j in range(_UNROLL):
                v = rbuf[pl.ds(off + j * _L, _L)]
                pos = iota + (flat_row + off + j * _L)
                cand = jnp.where(v == mvec, pos, _BIG_I32)
                if j % 2 == 0:
                    p0 = jnp.minimum(p0, cand)
                else:
                    p1 = jnp.minimum(p1, cand)
            return (p0, p1)

        pmin = lax.fori_loop(0, _NITER, rbody, pmin)

    flat = allmin(jnp.minimum(pmin[0], pmin[1]))[0]

    stage_v[...] = jnp.broadcast_to(m_best, (_L,))
    stage_i[...] = jnp.broadcast_to(flat, (_L,))
    pltpu.sync_copy(stage_v, outv_hbm.at[wid])
    pltpu.sync_copy(stage_i, outi_hbm.at[wid])


def _tc_partial_body(x_ref, outm_ref, outi_ref, m_s, i_s):
    k = pl.program_id(0)

    @pl.when(k == 0)
    def _():
        m_s[0] = jnp.float32(float("-inf"))
        i_s[0] = jnp.int32(0)

    xb = x_ref[...]
    m_b = jnp.max(xb)

    @pl.when(m_b > m_s[0])
    def _():
        mrow = jnp.max(xb, axis=1)
        riota = jax.lax.broadcasted_iota(jnp.int32, (_TC_BR,), 0)
        r0 = jnp.min(jnp.where(mrow == m_b, riota, _BIG_I32))
        row = x_ref[pl.ds(r0, 1), :]
        ciota = jax.lax.broadcasted_iota(jnp.int32, (1, _N1), 1)
        c0 = jnp.min(jnp.where(row == m_b, ciota, _BIG_I32))
        m_s[0] = m_b
        i_s[0] = (k * _TC_BR + r0) * _N1 + c0

    @pl.when(k == _TC_ROWS // _TC_BR - 1)
    def _():
        outm_ref[0] = m_s[0]
        outi_ref[0] = i_s[0]


_tc_partial = pl.pallas_call(
    _tc_partial_body,
    grid=(_TC_ROWS // _TC_BR,),
    in_specs=[pl.BlockSpec((_TC_BR, _N1), lambda k: (k, 0))],
    out_specs=[
        pl.BlockSpec(memory_space=pltpu.SMEM),
        pl.BlockSpec(memory_space=pltpu.SMEM),
    ],
    out_shape=[
        jax.ShapeDtypeStruct((1,), jnp.float32),
        jax.ShapeDtypeStruct((1,), jnp.int32),
    ],
    scratch_shapes=[
        pltpu.SMEM((1,), jnp.float32),
        pltpu.SMEM((1,), jnp.int32),
    ],
    compiler_params=pltpu.CompilerParams(
        dimension_semantics=("arbitrary",)),
)


def _combine_body(v_ref, i_ref, tm_ref, ti_ref, o_ref):
    vals = v_ref[...]
    idxs = i_ref[...]
    m = jnp.maximum(jnp.max(vals), tm_ref[0])
    cand = jnp.where(vals == m, idxs, _BIG_I32)
    idx = jnp.min(cand)
    idx = jnp.minimum(idx, jnp.where(tm_ref[0] == m, ti_ref[0], _BIG_I32))
    o_ref[0] = idx // _N0
    o_ref[1] = idx % _N1


_combine = pl.pallas_call(
    _combine_body,
    out_shape=jax.ShapeDtypeStruct((2,), jnp.int32),
    in_specs=[
        pl.BlockSpec(memory_space=pltpu.VMEM),
        pl.BlockSpec(memory_space=pltpu.VMEM),
        pl.BlockSpec(memory_space=pltpu.SMEM),
        pl.BlockSpec(memory_space=pltpu.SMEM),
    ],
    out_specs=pl.BlockSpec(memory_space=pltpu.SMEM),
)


def kernel(inputs):
    vals, idxs = _sc_partial_argmax(inputs)
    tcm, tci = _tc_partial(inputs)
    return _combine(vals, idxs, tcm, tci)


# speedup vs baseline: 1.0205x; 1.0205x over previous
import functools

import jax
import jax.numpy as jnp
from jax import lax
from jax.experimental import pallas as pl
from jax.experimental.pallas import tpu as pltpu
from jax.experimental.pallas import tpu_sc as plsc

_N0 = 8192
_N1 = 4096

_NC = 2
_NS = 16
_NW = _NC * _NS
_L = 16

_TC_ROWS = 4608
_SC_ROWS = _N0 - _TC_ROWS
_TC_BR = 512
_ROWS_W = _SC_ROWS // _NW
_CROWS = 1
_NBUF = 16
_NCHUNK = _ROWS_W // _CROWS
_UNROLL = 8
_NCHAIN = 4
_NITER = _N1 // (_L * _UNROLL)

_BIG_I32 = 2**31 - 1

_mesh = plsc.VectorSubcoreMesh(core_axis_name="c", subcore_axis_name="s")


@functools.partial(
    pl.kernel,
    mesh=_mesh,
    out_type=[
        jax.ShapeDtypeStruct((_NW, _L), jnp.float32),
        jax.ShapeDtypeStruct((_NW, _L), jnp.int32),
    ],
    scratch_types=[
        pltpu.VMEM((_NBUF, _CROWS, _N1), jnp.float32),
        pltpu.VMEM((_L,), jnp.float32),
        pltpu.VMEM((_L,), jnp.int32),
    ] + [pltpu.SemaphoreType.DMA] * _NBUF,
    compiler_params=pltpu.CompilerParams(use_tc_tiling_on_sc=True),
)
def _sc_partial_argmax(x_hbm, outv_hbm, outi_hbm, buf, stage_v, stage_i,
                       *sems):
    wid = lax.axis_index("s") * _NC + lax.axis_index("c")
    row_base = _TC_ROWS + wid * _ROWS_W
    iota = lax.iota(jnp.int32, _L)

    def allmax(x):
        for s in (8, 4, 2, 1):
            x = jnp.maximum(x, x.at[iota ^ s].get(mode="promise_in_bounds"))
        return x

    def allmin(x):
        for s in (8, 4, 2, 1):
            x = jnp.minimum(x, x.at[iota ^ s].get(mode="promise_in_bounds"))
        return x

    def start(c, slot):
        return pltpu.async_copy(
            x_hbm.at[pl.ds(row_base + c * _CROWS, _CROWS)], buf.at[slot],
            sems[slot])

    def wait_for(c, slot):
        pltpu.make_async_copy(
            x_hbm.at[pl.ds(row_base + c * _CROWS, _CROWS)], buf.at[slot],
            sems[slot]).wait()

    ninf = jnp.full((_L,), float("-inf"), jnp.float32)

    for c0 in range(_NBUF):
        start(c0, c0)

    def chunk_max(slot):
        acc = (ninf,) * _NCHAIN
        for r in range(_CROWS):
            rbuf = buf.at[slot, r]

            def mbody(i, accs, rbuf=rbuf):
                accs = list(accs)
                off = i * (_L * _UNROLL)
                for j in range(_UNROLL):
                    v = rbuf[pl.ds(off + j * _L, _L)]
                    k = j % _NCHAIN
                    accs[k] = jnp.maximum(accs[k], v)
                return tuple(accs)

            acc = lax.fori_loop(0, _NITER, mbody, acc)
        acc = list(acc)
        n = _NCHAIN
        while n > 1:
            n //= 2
            for k in range(n):
                acc[k] = jnp.maximum(acc[k], acc[k + n])
        return acc[0]

    zeros = jnp.zeros((_L,), jnp.int32)

    def cbody(k, carry):
        bestv, bestc = carry
        for slot in range(_NBUF):
            c = _NBUF * k + slot
            wait_for(c, slot)
            cm = chunk_max(slot)
            nxt = c + _NBUF

            @pl.when(nxt < _NCHUNK)
            def _():
                start(nxt, slot)

            better = cm > bestv
            bestc = jnp.where(better, c, bestc)
            bestv = jnp.maximum(bestv, cm)
        return (bestv, bestc)

    bestv, bestc = lax.fori_loop(0, _NCHUNK // _NBUF, cbody, (ninf, zeros))

    m_best = allmax(bestv)[0]
    mvec = jnp.broadcast_to(m_best, (_L,))
    c_best = allmin(jnp.where(bestv == mvec, bestc, _BIG_I32))[0]

    win_row = row_base + c_best * _CROWS
    pltpu.sync_copy(x_hbm.at[pl.ds(win_row, _CROWS)], buf.at[0])
    mvec = jnp.broadcast_to(m_best, (_L,))
    big = jnp.full((_L,), _BIG_I32, jnp.int32)

    pmin = (big, big)
    for r in range(_CROWS):
        rbuf = buf.at[0, r]
        flat_row = (win_row + r) * _N1

        def rbody(i, pmins, rbuf=rbuf, flat_row=flat_row):
            p0, p1 = pmins
            off = i * (_L * _UNROLL)
            for j in range(_UNROLL):
                v = rbuf[pl.ds(off + j * _L, _L)]
                pos = iota + (flat_row + off + j * _L)
                cand = jnp.where(v == mvec, pos, _BIG_I32)
                if j % 2 == 0:
                    p0 = jnp.minimum(p0, cand)
                else:
                    p1 = jnp.minimum(p1, cand)
            return (p0, p1)

        pmin = lax.fori_loop(0, _NITER, rbody, pmin)

    flat = allmin(jnp.minimum(pmin[0], pmin[1]))[0]

    stage_v[...] = jnp.broadcast_to(m_best, (_L,))
    stage_i[...] = jnp.broadcast_to(flat, (_L,))
    pltpu.sync_copy(stage_v, outv_hbm.at[wid])
    pltpu.sync_copy(stage_i, outi_hbm.at[wid])


def _tc_partial_body(x_ref, outm_ref, outi_ref, m_s, i_s):
    k = pl.program_id(0)

    @pl.when(k == 0)
    def _():
        m_s[0] = jnp.float32(float("-inf"))
        i_s[0] = jnp.int32(0)

    xb = x_ref[...]
    m_b = jnp.max(xb)

    @pl.when(m_b > m_s[0])
    def _():
        mrow = jnp.max(xb, axis=1)
        riota = jax.lax.broadcasted_iota(jnp.int32, (_TC_BR,), 0)
        r0 = jnp.min(jnp.where(mrow == m_b, riota, _BIG_I32))
        row = x_ref[pl.ds(r0, 1), :]
        ciota = jax.lax.broadcasted_iota(jnp.int32, (1, _N1), 1)
        c0 = jnp.min(jnp.where(row == m_b, ciota, _BIG_I32))
        m_s[0] = m_b
        i_s[0] = (k * _TC_BR + r0) * _N1 + c0

    @pl.when(k == _TC_ROWS // _TC_BR - 1)
    def _():
        outm_ref[0] = m_s[0]
        outi_ref[0] = i_s[0]


_tc_partial = pl.pallas_call(
    _tc_partial_body,
    grid=(_TC_ROWS // _TC_BR,),
    in_specs=[pl.BlockSpec((_TC_BR, _N1), lambda k: (k, 0))],
    out_specs=[
        pl.BlockSpec(memory_space=pltpu.SMEM),
        pl.BlockSpec(memory_space=pltpu.SMEM),
    ],
    out_shape=[
        jax.ShapeDtypeStruct((1,), jnp.float32),
        jax.ShapeDtypeStruct((1,), jnp.int32),
    ],
    scratch_shapes=[
        pltpu.SMEM((1,), jnp.float32),
        pltpu.SMEM((1,), jnp.int32),
    ],
    compiler_params=pltpu.CompilerParams(
        dimension_semantics=("arbitrary",)),
)


def _combine_body(v_ref, i_ref, tm_ref, ti_ref, o_ref):
    vals = v_ref[...]
    idxs = i_ref[...]
    m = jnp.maximum(jnp.max(vals), tm_ref[0])
    cand = jnp.where(vals == m, idxs, _BIG_I32)
    idx = jnp.min(cand)
    idx = jnp.minimum(idx, jnp.where(tm_ref[0] == m, ti_ref[0], _BIG_I32))
    o_ref[0] = idx // _N0
    o_ref[1] = idx % _N1


_combine = pl.pallas_call(
    _combine_body,
    out_shape=jax.ShapeDtypeStruct((2,), jnp.int32),
    in_specs=[
        pl.BlockSpec(memory_space=pltpu.VMEM),
        pl.BlockSpec(memory_space=pltpu.VMEM),
        pl.BlockSpec(memory_space=pltpu.SMEM),
        pl.BlockSpec(memory_space=pltpu.SMEM),
    ],
    out_specs=pl.BlockSpec(memory_space=pltpu.SMEM),
)


def kernel(inputs):
    vals, idxs = _sc_partial_argmax(inputs)
    tcm, tci = _tc_partial(inputs)
    return _combine(vals, idxs, tcm, tci)


# speedup vs baseline: 1.0209x; 1.0004x over previous
import functools

import jax
import jax.numpy as jnp
from jax import lax
from jax.experimental import pallas as pl
from jax.experimental.pallas import tpu as pltpu
from jax.experimental.pallas import tpu_sc as plsc

_N0 = 8192
_N1 = 4096

_NC = 2
_NS = 16
_NW = _NC * _NS
_L = 16

_TC_ROWS = 4608
_SC_ROWS = _N0 - _TC_ROWS
_TC_BR = 512
_ROWS_W = _SC_ROWS // _NW
_CROWS = 1
_NBUF = 16
_NCHUNK = _ROWS_W // _CROWS
_UNROLL = 4
_NCHAIN = 4
_NITER = _N1 // (_L * _UNROLL)

_BIG_I32 = 2**31 - 1

_mesh = plsc.VectorSubcoreMesh(core_axis_name="c", subcore_axis_name="s")


@functools.partial(
    pl.kernel,
    mesh=_mesh,
    out_type=[
        jax.ShapeDtypeStruct((_NW, _L), jnp.float32),
        jax.ShapeDtypeStruct((_NW, _L), jnp.int32),
    ],
    scratch_types=[
        pltpu.VMEM((_NBUF, _CROWS, _N1), jnp.float32),
        pltpu.VMEM((_L,), jnp.float32),
        pltpu.VMEM((_L,), jnp.int32),
    ] + [pltpu.SemaphoreType.DMA] * _NBUF,
    compiler_params=pltpu.CompilerParams(use_tc_tiling_on_sc=True),
)
def _sc_partial_argmax(x_hbm, outv_hbm, outi_hbm, buf, stage_v, stage_i,
                       *sems):
    wid = lax.axis_index("s") * _NC + lax.axis_index("c")
    row_base = _TC_ROWS + wid * _ROWS_W
    iota = lax.iota(jnp.int32, _L)

    def allmax(x):
        for s in (8, 4, 2, 1):
            x = jnp.maximum(x, x.at[iota ^ s].get(mode="promise_in_bounds"))
        return x

    def allmin(x):
        for s in (8, 4, 2, 1):
            x = jnp.minimum(x, x.at[iota ^ s].get(mode="promise_in_bounds"))
        return x

    def start(c, slot):
        return pltpu.async_copy(
            x_hbm.at[pl.ds(row_base + c * _CROWS, _CROWS)], buf.at[slot],
            sems[slot])

    def wait_for(c, slot):
        pltpu.make_async_copy(
            x_hbm.at[pl.ds(row_base + c * _CROWS, _CROWS)], buf.at[slot],
            sems[slot]).wait()

    ninf = jnp.full((_L,), float("-inf"), jnp.float32)

    for c0 in range(_NBUF):
        start(c0, c0)

    def chunk_max(slot):
        acc = (ninf,) * _NCHAIN
        for r in range(_CROWS):
            rbuf = buf.at[slot, r]

            def mbody(i, accs, rbuf=rbuf):
                accs = list(accs)
                off = i * (_L * _UNROLL)
                for j in range(_UNROLL):
                    v = rbuf[pl.ds(off + j * _L, _L)]
                    k = j % _NCHAIN
                    accs[k] = jnp.maximum(accs[k], v)
                return tuple(accs)

            acc = lax.fori_loop(0, _NITER, mbody, acc)
        acc = list(acc)
        n = _NCHAIN
        while n > 1:
            n //= 2
            for k in range(n):
                acc[k] = jnp.maximum(acc[k], acc[k + n])
        return acc[0]

    zeros = jnp.zeros((_L,), jnp.int32)

    def cbody(k, carry):
        bestv, bestc = carry
        for slot in range(_NBUF):
            c = _NBUF * k + slot
            wait_for(c, slot)
            cm = chunk_max(slot)
            nxt = c + _NBUF

            @pl.when(nxt < _NCHUNK)
            def _():
                start(nxt, slot)

            better = cm > bestv
            bestc = jnp.where(better, c, bestc)
            bestv = jnp.maximum(bestv, cm)
        return (bestv, bestc)

    bestv, bestc = lax.fori_loop(0, _NCHUNK // _NBUF, cbody, (ninf, zeros))

    m_best = allmax(bestv)[0]
    mvec = jnp.broadcast_to(m_best, (_L,))
    c_best = allmin(jnp.where(bestv == mvec, bestc, _BIG_I32))[0]

    win_row = row_base + c_best * _CROWS
    pltpu.sync_copy(x_hbm.at[pl.ds(win_row, _CROWS)], buf.at[0])
    mvec = jnp.broadcast_to(m_best, (_L,))
    big = jnp.full((_L,), _BIG_I32, jnp.int32)

    pmin = (big, big)
    for r in range(_CROWS):
        rbuf = buf.at[0, r]
        flat_row = (win_row + r) * _N1

        def rbody(i, pmins, rbuf=rbuf, flat_row=flat_row):
            p0, p1 = pmins
            off = i * (_L * _UNROLL)
            for j in range(_UNROLL):
                v = rbuf[pl.ds(off + j * _L, _L)]
                pos = iota + (flat_row + off + j * _L)
                cand = jnp.where(v == mvec, pos, _BIG_I32)
                if j % 2 == 0:
                    p0 = jnp.minimum(p0, cand)
                else:
                    p1 = jnp.minimum(p1, cand)
            return (p0, p1)

        pmin = lax.fori_loop(0, _NITER, rbody, pmin)

    flat = allmin(jnp.minimum(pmin[0], pmin[1]))[0]

    stage_v[...] = jnp.broadcast_to(m_best, (_L,))
    stage_i[...] = jnp.broadcast_to(flat, (_L,))
    pltpu.sync_copy(stage_v, outv_hbm.at[wid])
    pltpu.sync_copy(stage_i, outi_hbm.at[wid])


def _tc_partial_body(x_ref, outm_ref, outi_ref, m_s, i_s):
    k = pl.program_id(0)

    @pl.when(k == 0)
    def _():
        m_s[0] = jnp.float32(float("-inf"))
        i_s[0] = jnp.int32(0)

    xb = x_ref[...]
    m_b = jnp.max(xb)

    @pl.when(m_b > m_s[0])
    def _():
        mrow = jnp.max(xb, axis=1)
        riota = jax.lax.broadcasted_iota(jnp.int32, (_TC_BR,), 0)
        r0 = jnp.min(jnp.where(mrow == m_b, riota, _BIG_I32))
        row = x_ref[pl.ds(r0, 1), :]
        ciota = jax.lax.broadcasted_iota(jnp.int32, (1, _N1), 1)
        c0 = jnp.min(jnp.where(row == m_b, ciota, _BIG_I32))
        m_s[0] = m_b
        i_s[0] = (k * _TC_BR + r0) * _N1 + c0

    @pl.when(k == _TC_ROWS // _TC_BR - 1)
    def _():
        outm_ref[0] = m_s[0]
        outi_ref[0] = i_s[0]


_tc_partial = pl.pallas_call(
    _tc_partial_body,
    grid=(_TC_ROWS // _TC_BR,),
    in_specs=[pl.BlockSpec((_TC_BR, _N1), lambda k: (k, 0))],
    out_specs=[
        pl.BlockSpec(memory_space=pltpu.SMEM),
        pl.BlockSpec(memory_space=pltpu.SMEM),
    ],
    out_shape=[
        jax.ShapeDtypeStruct((1,), jnp.float32),
        jax.ShapeDtypeStruct((1,), jnp.int32),
    ],
    scratch_shapes=[
        pltpu.SMEM((1,), jnp.float32),
        pltpu.SMEM((1,), jnp.int32),
    ],
    compiler_params=pltpu.CompilerParams(
        dimension_semantics=("arbitrary",)),
)


def _combine_body(v_ref, i_ref, tm_ref, ti_ref, o_ref):
    vals = v_ref[...]
    idxs = i_ref[...]
    m = jnp.maximum(jnp.max(vals), tm_ref[0])
    cand = jnp.where(vals == m, idxs, _BIG_I32)
    idx = jnp.min(cand)
    idx = jnp.minimum(idx, jnp.where(tm_ref[0] == m, ti_ref[0], _BIG_I32))
    o_ref[0] = idx // _N0
    o_ref[1] = idx % _N1


_combine = pl.pallas_call(
    _combine_body,
    out_shape=jax.ShapeDtypeStruct((2,), jnp.int32),
    in_specs=[
        pl.BlockSpec(memory_space=pltpu.VMEM),
        pl.BlockSpec(memory_space=pltpu.VMEM),
        pl.BlockSpec(memory_space=pltpu.SMEM),
        pl.BlockSpec(memory_space=pltpu.SMEM),
    ],
    out_specs=pl.BlockSpec(memory_space=pltpu.SMEM),
)


def kernel(inputs):
    vals, idxs = _sc_partial_argmax(inputs)
    tcm, tci = _tc_partial(inputs)
    return _combine(vals, idxs, tcm, tci)


# speedup vs baseline: 1.0264x; 1.0054x over previous
import functools

import jax
import jax.numpy as jnp
from jax import lax
from jax.experimental import pallas as pl
from jax.experimental.pallas import tpu as pltpu
from jax.experimental.pallas import tpu_sc as plsc

_N0 = 8192
_N1 = 4096

_NC = 2
_NS = 16
_NW = _NC * _NS
_L = 16

_TC_ROWS = 4480
_SC_ROWS = _N0 - _TC_ROWS
_TC_BR = 560
_ROWS_W = _SC_ROWS // _NW
_CROWS = 1
_NBUF = 16
_NCHUNK = _ROWS_W // _CROWS
_UNROLL = 8
_NCHAIN = 4
_NITER = _N1 // (_L * _UNROLL)

_BIG_I32 = 2**31 - 1

_mesh = plsc.VectorSubcoreMesh(core_axis_name="c", subcore_axis_name="s")


@functools.partial(
    pl.kernel,
    mesh=_mesh,
    out_type=[
        jax.ShapeDtypeStruct((_NW, _L), jnp.float32),
        jax.ShapeDtypeStruct((_NW, _L), jnp.int32),
    ],
    scratch_types=[
        pltpu.VMEM((_NBUF, _CROWS, _N1), jnp.float32),
        pltpu.VMEM((_L,), jnp.float32),
        pltpu.VMEM((_L,), jnp.int32),
    ] + [pltpu.SemaphoreType.DMA] * _NBUF,
    compiler_params=pltpu.CompilerParams(use_tc_tiling_on_sc=True),
)
def _sc_partial_argmax(x_hbm, outv_hbm, outi_hbm, buf, stage_v, stage_i,
                       *sems):
    wid = lax.axis_index("s") * _NC + lax.axis_index("c")
    row_base = _TC_ROWS + wid * _ROWS_W
    iota = lax.iota(jnp.int32, _L)

    def allmax(x):
        for s in (8, 4, 2, 1):
            x = jnp.maximum(x, x.at[iota ^ s].get(mode="promise_in_bounds"))
        return x

    def allmin(x):
        for s in (8, 4, 2, 1):
            x = jnp.minimum(x, x.at[iota ^ s].get(mode="promise_in_bounds"))
        return x

    def start(c, slot):
        return pltpu.async_copy(
            x_hbm.at[pl.ds(row_base + c * _CROWS, _CROWS)], buf.at[slot],
            sems[slot])

    def wait_for(c, slot):
        pltpu.make_async_copy(
            x_hbm.at[pl.ds(row_base + c * _CROWS, _CROWS)], buf.at[slot],
            sems[slot]).wait()

    ninf = jnp.full((_L,), float("-inf"), jnp.float32)

    for c0 in range(_NBUF):
        start(c0, c0)

    def chunk_max(slot):
        acc = (ninf,) * _NCHAIN
        for r in range(_CROWS):
            rbuf = buf.at[slot, r]

            def mbody(i, accs, rbuf=rbuf):
                accs = list(accs)
                off = i * (_L * _UNROLL)
                for j in range(_UNROLL):
                    v = rbuf[pl.ds(off + j * _L, _L)]
                    k = j % _NCHAIN
                    accs[k] = jnp.maximum(accs[k], v)
                return tuple(accs)

            acc = lax.fori_loop(0, _NITER, mbody, acc)
        acc = list(acc)
        n = _NCHAIN
        while n > 1:
            n //= 2
            for k in range(n):
                acc[k] = jnp.maximum(acc[k], acc[k + n])
        return acc[0]

    zeros = jnp.zeros((_L,), jnp.int32)

    def cbody(k, carry):
        bestv, bestc = carry
        for slot in range(_NBUF):
            c = _NBUF * k + slot
            wait_for(c, slot)
            cm = chunk_max(slot)
            nxt = c + _NBUF

            @pl.when(nxt < _NCHUNK)
            def _():
                start(nxt, slot)

            better = cm > bestv
            bestc = jnp.where(better, c, bestc)
            bestv = jnp.maximum(bestv, cm)
        return (bestv, bestc)

    bestv, bestc = lax.fori_loop(0, _NCHUNK // _NBUF, cbody, (ninf, zeros))
    for slot in range(_NCHUNK % _NBUF):
        c = (_NCHUNK // _NBUF) * _NBUF + slot
        wait_for(c, slot)
        cm = chunk_max(slot)
        better = cm > bestv
        bestc = jnp.where(better, c, bestc)
        bestv = jnp.maximum(bestv, cm)

    m_best = allmax(bestv)[0]
    mvec = jnp.broadcast_to(m_best, (_L,))
    c_best = allmin(jnp.where(bestv == mvec, bestc, _BIG_I32))[0]

    win_row = row_base + c_best * _CROWS
    pltpu.sync_copy(x_hbm.at[pl.ds(win_row, _CROWS)], buf.at[0])
    mvec = jnp.broadcast_to(m_best, (_L,))
    big = jnp.full((_L,), _BIG_I32, jnp.int32)

    pmin = (big, big)
    for r in range(_CROWS):
        rbuf = buf.at[0, r]
        flat_row = (win_row + r) * _N1

        def rbody(i, pmins, rbuf=rbuf, flat_row=flat_row):
            p0, p1 = pmins
            off = i * (_L * _UNROLL)
            for j in range(_UNROLL):
                v = rbuf[pl.ds(off + j * _L, _L)]
                pos = iota + (flat_row + off + j * _L)
                cand = jnp.where(v == mvec, pos, _BIG_I32)
                if j % 2 == 0:
                    p0 = jnp.minimum(p0, cand)
                else:
                    p1 = jnp.minimum(p1, cand)
            return (p0, p1)

        pmin = lax.fori_loop(0, _NITER, rbody, pmin)

    flat = allmin(jnp.minimum(pmin[0], pmin[1]))[0]

    stage_v[...] = jnp.broadcast_to(m_best, (_L,))
    stage_i[...] = jnp.broadcast_to(flat, (_L,))
    pltpu.sync_copy(stage_v, outv_hbm.at[wid])
    pltpu.sync_copy(stage_i, outi_hbm.at[wid])


def _tc_partial_body(x_ref, outm_ref, outi_ref, m_s, i_s):
    k = pl.program_id(0)

    @pl.when(k == 0)
    def _():
        m_s[0] = jnp.float32(float("-inf"))
        i_s[0] = jnp.int32(0)

    xb = x_ref[...]
    m_b = jnp.max(xb)

    @pl.when(m_b > m_s[0])
    def _():
        mrow = jnp.max(xb, axis=1)
        riota = jax.lax.broadcasted_iota(jnp.int32, (_TC_BR,), 0)
        r0 = jnp.min(jnp.where(mrow == m_b, riota, _BIG_I32))
        row = x_ref[pl.ds(r0, 1), :]
        ciota = jax.lax.broadcasted_iota(jnp.int32, (1, _N1), 1)
        c0 = jnp.min(jnp.where(row == m_b, ciota, _BIG_I32))
        m_s[0] = m_b
        i_s[0] = (k * _TC_BR + r0) * _N1 + c0

    @pl.when(k == _TC_ROWS // _TC_BR - 1)
    def _():
        outm_ref[0] = m_s[0]
        outi_ref[0] = i_s[0]


_tc_partial = pl.pallas_call(
    _tc_partial_body,
    grid=(_TC_ROWS // _TC_BR,),
    in_specs=[pl.BlockSpec((_TC_BR, _N1), lambda k: (k, 0))],
    out_specs=[
        pl.BlockSpec(memory_space=pltpu.SMEM),
        pl.BlockSpec(memory_space=pltpu.SMEM),
    ],
    out_shape=[
        jax.ShapeDtypeStruct((1,), jnp.float32),
        jax.ShapeDtypeStruct((1,), jnp.int32),
    ],
    scratch_shapes=[
        pltpu.SMEM((1,), jnp.float32),
        pltpu.SMEM((1,), jnp.int32),
    ],
    compiler_params=pltpu.CompilerParams(
        dimension_semantics=("arbitrary",)),
)


def _combine_body(v_ref, i_ref, tm_ref, ti_ref, o_ref):
    vals = v_ref[...]
    idxs = i_ref[...]
    m = jnp.maximum(jnp.max(vals), tm_ref[0])
    cand = jnp.where(vals == m, idxs, _BIG_I32)
    idx = jnp.min(cand)
    idx = jnp.minimum(idx, jnp.where(tm_ref[0] == m, ti_ref[0], _BIG_I32))
    o_ref[0] = idx // _N0
    o_ref[1] = idx % _N1


_combine = pl.pallas_call(
    _combine_body,
    out_shape=jax.ShapeDtypeStruct((2,), jnp.int32),
    in_specs=[
        pl.BlockSpec(memory_space=pltpu.VMEM),
        pl.BlockSpec(memory_space=pltpu.VMEM),
        pl.BlockSpec(memory_space=pltpu.SMEM),
        pl.BlockSpec(memory_space=pltpu.SMEM),
    ],
    out_specs=pl.BlockSpec(memory_space=pltpu.SMEM),
)


def kernel(inputs):
    vals, idxs = _sc_partial_argmax(inputs)
    tcm, tci = _tc_partial(inputs)
    return _combine(vals, idxs, tcm, tci)
